# baseline (device time: 247317 ns/iter reference)
import jax
import jax.numpy as jnp
from jax import lax
from jax.experimental import pallas as pl
from jax.experimental.pallas import tpu as pltpu

N_DEV = 8
E_PER = 2
F_TILES = 2


def kernel(x, assign, W1, W2):
    t, d = x.shape
    e_per, _, f = W1.shape
    assert e_per == E_PER
    pad = 128
    dc = d + pad
    th = t // 2
    fw = f // F_TILES

    x_bf = x.astype(jnp.bfloat16)
    a_col = assign.astype(jnp.bfloat16).reshape(t, 1)
    a_pad = jnp.zeros((t, pad - 1), jnp.bfloat16)
    xcat = jnp.concatenate([x_bf, a_col, a_pad], axis=1)
    w1_cat = jnp.transpose(W1.astype(jnp.bfloat16), (1, 0, 2)).reshape(
        d, E_PER * f)
    w2_cat = W2.astype(jnp.bfloat16).reshape(E_PER * f, d)

    def body(x_ref, w1_ref, w2_ref, out_ref,
             xs_r, xs_l, csend_r, csend_l, crecv_r, crecv_l,
             ag_s_r, ag_r_r, ag_s_l, ag_r_l,
             rs_s_r, rs_r_r, rs_s_l, rs_r_l,
             credit_r, credit_l):
        my = lax.axis_index("i")
        left = (my + N_DEV - 1) % N_DEV
        right = (my + 1) % N_DEV

        rings = (
            (xs_r, csend_r, crecv_r, ag_s_r, ag_r_r, rs_s_r, rs_r_r,
             credit_r, right, left),
            (xs_l, csend_l, crecv_l, ag_s_l, ag_r_l, rs_s_l, rs_r_l,
             credit_l, left, right),
        )

        def mk_ag(ring, h):
            xs, _, _, ag_s, ag_r, _, _, _, dst, _ = ring
            return pltpu.make_async_remote_copy(
                src_ref=xs.at[h], dst_ref=xs.at[h + 1],
                send_sem=ag_s.at[h], recv_sem=ag_r.at[h],
                device_id=(dst,), device_id_type=pl.DeviceIdType.MESH,
            )

        def mk_rs(ring, k):
            _, csend, crecv, _, _, rs_s, rs_r, _, dst, _ = ring
            return pltpu.make_async_remote_copy(
                src_ref=csend.at[k % 2], dst_ref=crecv.at[k % 2],
                send_sem=rs_s.at[k], recv_sem=rs_r.at[k],
                device_id=(dst,), device_id_type=pl.DeviceIdType.MESH,
            )

        barrier = pltpu.get_barrier_semaphore()
        for nbr in (left, right):
            pl.semaphore_signal(
                barrier, inc=1,
                device_id=(nbr,), device_id_type=pl.DeviceIdType.MESH,
            )
        pl.semaphore_wait(barrier, 2)

        xs_r[0] = x_ref[0:th, :]
        xs_l[0] = x_ref[th:t, :]

        for ring in rings:
            mk_ag(ring, 0).start()

        n_tiles = E_PER * F_TILES
        tw = E_PER * f // n_tiles

        def partial_rows(xs, slot):
            chunk = xs[slot]
            xc = chunk[:, 0:d]
            a = chunk[:, d:d + 1]
            y = jnp.zeros((th, d), jnp.float32)
            for ft in range(n_tiles):
                e_val = (my * E_PER + ft // F_TILES).astype(jnp.bfloat16)
                c0 = ft * tw
                h1 = jnp.dot(xc, w1_ref[:, c0:c0 + tw],
                             preferred_element_type=jnp.float32)
                h1 = jnp.maximum(h1, 0.0).astype(jnp.bfloat16)
                h1 = jnp.where(a == e_val, h1, jnp.bfloat16(0))
                y = y + jnp.dot(h1, w2_ref[c0:c0 + tw, :],
                                preferred_element_type=jnp.float32)
            return y

        def hop(k, carry):
            kn = jnp.minimum(k + 1, N_DEV - 2)
            kp2 = jnp.maximum(k - 2, 0)
            kp1 = jnp.maximum(k - 1, 0)

            for ring in rings:
                mk_ag(ring, k).wait_recv()

            @pl.when(k < N_DEV - 2)
            def _():
                for ring in rings:
                    mk_ag(ring, kn).start()

            @pl.when(k >= 2)
            def _():
                for ring in rings:
                    mk_rs(ring, kp2).wait_send()

            for ring in rings:
                xs, csend = ring[0], ring[1]
                csend[k % 2] = partial_rows(xs, k + 1).astype(jnp.bfloat16)

            @pl.when(k >= 1)
            def _():
                for ring in rings:
                    csend, crecv = ring[1], ring[2]
                    mk_rs(ring, kp1).wait_recv()
                    csend[k % 2] = csend[k % 2] + crecv[kp1 % 2]

            @pl.when(jnp.logical_and(k >= 1, k <= N_DEV - 3))
            def _():
                for ring in rings:
                    pl.semaphore_signal(
                        ring[7], inc=1,
                        device_id=(ring[9],),
                        device_id_type=pl.DeviceIdType.MESH,
                    )

            @pl.when(k >= 2)
            def _():
                for ring in rings:
                    pl.semaphore_wait(ring[7], 1)

            for ring in rings:
                mk_rs(ring, k).start()
            return carry

        lax.fori_loop(0, N_DEV - 1, hop, 0)

        for ring, r0 in ((rings[0], 0), (rings[1], th)):
            xs, crecv = ring[0], ring[2]
            mk_rs(ring, N_DEV - 2).wait_recv()
            acc = partial_rows(xs, 0) + crecv[0].astype(jnp.float32)
            out_ref[r0:r0 + th, :] = acc

        def drain(h, carry):
            for ring in rings:
                mk_ag(ring, h).wait_send()
            return carry

        lax.fori_loop(0, N_DEV - 1, drain, 0)
        for ring in rings:
            mk_rs(ring, N_DEV - 3).wait_send()
            mk_rs(ring, N_DEV - 2).wait_send()

    return pl.pallas_call(
        body,
        out_shape=jax.ShapeDtypeStruct((t, d), jnp.float32),
        in_specs=[
            pl.BlockSpec(memory_space=pltpu.VMEM),
            pl.BlockSpec(memory_space=pltpu.VMEM),
            pl.BlockSpec(memory_space=pltpu.VMEM),
        ],
        out_specs=pl.BlockSpec(memory_space=pltpu.VMEM),
        scratch_shapes=[
            pltpu.VMEM((N_DEV, th, dc), jnp.bfloat16),
            pltpu.VMEM((N_DEV, th, dc), jnp.bfloat16),
            pltpu.VMEM((2, th, d), jnp.bfloat16),
            pltpu.VMEM((2, th, d), jnp.bfloat16),
            pltpu.VMEM((2, th, d), jnp.bfloat16),
            pltpu.VMEM((2, th, d), jnp.bfloat16),
            pltpu.SemaphoreType.DMA((N_DEV - 1,)),
            pltpu.SemaphoreType.DMA((N_DEV - 1,)),
            pltpu.SemaphoreType.DMA((N_DEV - 1,)),
            pltpu.SemaphoreType.DMA((N_DEV - 1,)),
            pltpu.SemaphoreType.DMA((N_DEV - 1,)),
            pltpu.SemaphoreType.DMA((N_DEV - 1,)),
            pltpu.SemaphoreType.DMA((N_DEV - 1,)),
            pltpu.SemaphoreType.DMA((N_DEV - 1,)),
            pltpu.SemaphoreType.REGULAR,
            pltpu.SemaphoreType.REGULAR,
        ],
        compiler_params=pltpu.CompilerParams(
            collective_id=0,
            vmem_limit_bytes=62 * 1024 * 1024,
        ),
    )(xcat, w1_cat, w2_cat)


# device time: 232977 ns/iter; 1.0616x vs baseline; 1.0616x over previous
import jax
import jax.numpy as jnp
from jax import lax
from jax.experimental import pallas as pl
from jax.experimental.pallas import tpu as pltpu

N_DEV = 8
E_PER = 2
F_TILES = 2


def kernel(x, assign, W1, W2):
    t, d = x.shape
    e_per, _, f = W1.shape
    assert e_per == E_PER
    pad = 128
    dc = d + pad
    th = t // 2
    fw = f // F_TILES

    x_bf = x.astype(jnp.bfloat16)
    a_col = assign.astype(jnp.bfloat16).reshape(t, 1)
    a_pad = jnp.zeros((t, pad - 1), jnp.bfloat16)
    xcat = jnp.concatenate([x_bf, a_col, a_pad], axis=1)
    w1_cat = jnp.transpose(W1.astype(jnp.bfloat16), (1, 0, 2)).reshape(
        d, E_PER * f)
    w2_cat = W2.astype(jnp.bfloat16).reshape(E_PER * f, d)

    def body(x_ref, w1_ref, w2_ref, out_ref,
             xs_r, xs_l, csend_r, csend_l, crecv_r, crecv_l,
             ag_s_r, ag_r_r, ag_s_l, ag_r_l,
             rs_s_r, rs_r_r, rs_s_l, rs_r_l,
             credit_r, credit_l):
        my = lax.axis_index("i")
        left = (my + N_DEV - 1) % N_DEV
        right = (my + 1) % N_DEV

        rings = (
            (xs_r, csend_r, crecv_r, ag_s_r, ag_r_r, rs_s_r, rs_r_r,
             credit_r, right, left),
            (xs_l, csend_l, crecv_l, ag_s_l, ag_r_l, rs_s_l, rs_r_l,
             credit_l, left, right),
        )

        def mk_ag(ring, h):
            xs, _, _, ag_s, ag_r, _, _, _, dst, _ = ring
            return pltpu.make_async_remote_copy(
                src_ref=xs.at[h], dst_ref=xs.at[h + 1],
                send_sem=ag_s.at[h], recv_sem=ag_r.at[h],
                device_id=(dst,), device_id_type=pl.DeviceIdType.MESH,
            )

        def mk_rs(ring, k):
            _, csend, crecv, _, _, rs_s, rs_r, _, dst, _ = ring
            return pltpu.make_async_remote_copy(
                src_ref=csend.at[k % 2], dst_ref=crecv.at[k % 2],
                send_sem=rs_s.at[k], recv_sem=rs_r.at[k],
                device_id=(dst,), device_id_type=pl.DeviceIdType.MESH,
            )

        barrier = pltpu.get_barrier_semaphore()
        for nbr in (left, right):
            pl.semaphore_signal(
                barrier, inc=1,
                device_id=(nbr,), device_id_type=pl.DeviceIdType.MESH,
            )
        pl.semaphore_wait(barrier, 2)

        xs_r[0] = x_ref[0:th, :]
        xs_l[0] = x_ref[th:t, :]

        for ring in rings:
            mk_ag(ring, 0).start()

        n_tiles = E_PER * F_TILES
        tw = E_PER * f // n_tiles

        def partial_rows(xs, slot):
            chunk = xs[slot]
            xc = chunk[:, 0:d]
            a = chunk[:, d:d + 1]
            y = jnp.zeros((th, d), jnp.float32)
            for ft in range(n_tiles):
                e_val = (my * E_PER + ft // F_TILES).astype(jnp.bfloat16)
                c0 = ft * tw
                h1 = jnp.dot(xc, w1_ref[:, c0:c0 + tw],
                             preferred_element_type=jnp.float32)
                h1 = jnp.maximum(h1, 0.0).astype(jnp.bfloat16)
                h1 = jnp.where(a == e_val, h1, jnp.bfloat16(0))
                y = y + jnp.dot(h1, w2_ref[c0:c0 + tw, :],
                                preferred_element_type=jnp.float32)
            return y

        out_ref[0:th, :] = partial_rows(xs_r, 0)
        out_ref[th:t, :] = partial_rows(xs_l, 0)

        def hop(k, carry):
            kn = jnp.minimum(k + 1, N_DEV - 2)
            kp2 = jnp.maximum(k - 2, 0)
            kp1 = jnp.maximum(k - 1, 0)

            for ring in rings:
                mk_ag(ring, k).wait_recv()

            @pl.when(k < N_DEV - 2)
            def _():
                for ring in rings:
                    mk_ag(ring, kn).start()

            @pl.when(k >= 2)
            def _():
                for ring in rings:
                    mk_rs(ring, kp2).wait_send()

            for ring in rings:
                xs, csend = ring[0], ring[1]
                csend[k % 2] = partial_rows(xs, k + 1).astype(jnp.bfloat16)

            @pl.when(k >= 1)
            def _():
                for ring in rings:
                    csend, crecv = ring[1], ring[2]
                    mk_rs(ring, kp1).wait_recv()
                    csend[k % 2] = csend[k % 2] + crecv[kp1 % 2]

            @pl.when(jnp.logical_and(k >= 1, k <= N_DEV - 3))
            def _():
                for ring in rings:
                    pl.semaphore_signal(
                        ring[7], inc=1,
                        device_id=(ring[9],),
                        device_id_type=pl.DeviceIdType.MESH,
                    )

            @pl.when(k >= 2)
            def _():
                for ring in rings:
                    pl.semaphore_wait(ring[7], 1)

            for ring in rings:
                mk_rs(ring, k).start()
            return carry

        lax.fori_loop(0, N_DEV - 1, hop, 0)

        for ring, r0 in ((rings[0], 0), (rings[1], th)):
            crecv = ring[2]
            mk_rs(ring, N_DEV - 2).wait_recv()
            out_ref[r0:r0 + th, :] = (
                out_ref[r0:r0 + th, :] + crecv[0].astype(jnp.float32))

        def drain(h, carry):
            for ring in rings:
                mk_ag(ring, h).wait_send()
            return carry

        lax.fori_loop(0, N_DEV - 1, drain, 0)
        for ring in rings:
            mk_rs(ring, N_DEV - 3).wait_send()
            mk_rs(ring, N_DEV - 2).wait_send()

    return pl.pallas_call(
        body,
        out_shape=jax.ShapeDtypeStruct((t, d), jnp.float32),
        in_specs=[
            pl.BlockSpec(memory_space=pltpu.VMEM),
            pl.BlockSpec(memory_space=pltpu.VMEM),
            pl.BlockSpec(memory_space=pltpu.VMEM),
        ],
        out_specs=pl.BlockSpec(memory_space=pltpu.VMEM),
        scratch_shapes=[
            pltpu.VMEM((N_DEV, th, dc), jnp.bfloat16),
            pltpu.VMEM((N_DEV, th, dc), jnp.bfloat16),
            pltpu.VMEM((2, th, d), jnp.bfloat16),
            pltpu.VMEM((2, th, d), jnp.bfloat16),
            pltpu.VMEM((2, th, d), jnp.bfloat16),
            pltpu.VMEM((2, th, d), jnp.bfloat16),
            pltpu.SemaphoreType.DMA((N_DEV - 1,)),
            pltpu.SemaphoreType.DMA((N_DEV - 1,)),
            pltpu.SemaphoreType.DMA((N_DEV - 1,)),
            pltpu.SemaphoreType.DMA((N_DEV - 1,)),
            pltpu.SemaphoreType.DMA((N_DEV - 1,)),
            pltpu.SemaphoreType.DMA((N_DEV - 1,)),
            pltpu.SemaphoreType.DMA((N_DEV - 1,)),
            pltpu.SemaphoreType.DMA((N_DEV - 1,)),
            pltpu.SemaphoreType.REGULAR,
            pltpu.SemaphoreType.REGULAR,
        ],
        compiler_params=pltpu.CompilerParams(
            collective_id=0,
            vmem_limit_bytes=62 * 1024 * 1024,
        ),
    )(xcat, w1_cat, w2_cat)
